# baseline (speedup 1.0000x reference)
import functools

import jax
import jax.numpy as jnp
from jax import lax
from jax.experimental import pallas as pl
from jax.experimental.pallas import tpu as pltpu

_EPS = 1e-5


def _fused_kernel(x_ref, wdw_ref, w2_ref, g1_ref, b1_ref, g2_ref, b2_ref,
                  y_ref,
                  xf_ref, xa_ref, xb_ref, hbuf_ref, st1_ref, a1c1_ref,
                  ms_ref, ss_ref, a2c2_ref,
                  *, bb, cin, ww, hw, padhw, inv_count):
    ph = pl.program_id(0)
    i = pl.program_id(1)

    @pl.when(ph == 0)
    def _phase0():
        @pl.when(i == 0)
        def _init():
            xf_ref[...] = jnp.zeros_like(xf_ref)
            st1_ref[...] = jnp.zeros_like(st1_ref)

        for b in range(bb):
            xf_ref[b, :, 2 * ww:2 * ww + hw] = x_ref[b].reshape(cin, hw)
        xfv = xf_ref[...]
        lane = lax.broadcasted_iota(jnp.int32, (1, padhw), 1) % ww
        xa_ref[...] = jnp.where(lane == ww - 1, 0.0, xfv)
        xb_ref[...] = jnp.where(lane == 0, 0.0, xfv)
        acc = None
        for kh in range(3):
            for kw in range(3):
                src = (xa_ref, xf_ref, xb_ref)[kw]
                s = (ww - 1) + kh * ww + kw
                t = wdw_ref[kh * 3 + kw] * src[:, :, s:s + hw]
                acc = t if acc is None else acc + t
        hbuf_ref[pl.ds(i * bb, bb)] = acc.astype(jnp.bfloat16)
        st1_ref[0] += jnp.sum(acc, axis=(0, 2), keepdims=True)[0]
        st1_ref[1] += jnp.sum(acc * acc, axis=(0, 2), keepdims=True)[0]

    @pl.when(ph == 1)
    def _phase1():
        @pl.when(i == 0)
        def _fold_bn1():
            mean = st1_ref[0] * inv_count
            var = jnp.maximum(st1_ref[1] * inv_count - mean * mean, 0.0)
            a1 = g1_ref[...] * lax.rsqrt(var + _EPS)
            a1c1_ref[0] = a1
            a1c1_ref[1] = b1_ref[...] - mean * a1
            ms_ref[...] = jnp.zeros_like(ms_ref)
            ss_ref[...] = jnp.zeros_like(ss_ref)

        g = jnp.clip(hbuf_ref[pl.ds(i * bb, bb)].astype(jnp.float32)
                     * a1c1_ref[0] + a1c1_ref[1], 0.0, 6.0)
        gq = g.astype(jnp.bfloat16)
        hbuf_ref[pl.ds(i * bb, bb)] = gq
        ss_ref[...] += jnp.sum(gq.astype(jnp.float32), axis=(0, 2),
                               keepdims=True)[0]
        gram = None
        for b in range(bb):
            p = lax.dot_general(gq[b], gq[b], (((1,), (1,)), ((), ())),
                                preferred_element_type=jnp.float32)
            gram = p if gram is None else gram + p
        ms_ref[...] += gram

    @pl.when(ph == 2)
    def _phase2():
        @pl.when(i == 0)
        def _fold_bn2():
            w2v = w2_ref[...].astype(jnp.float32)
            meanz = lax.dot_general(
                w2v, ss_ref[...], (((1,), (0,)), ((), ())),
                preferred_element_type=jnp.float32) * inv_count
            t = jnp.dot(w2v, ms_ref[...],
                        preferred_element_type=jnp.float32)
            ez2 = jnp.sum(t * w2v, axis=1, keepdims=True) * inv_count
            var = jnp.maximum(ez2 - meanz * meanz, 0.0)
            a2 = g2_ref[...] * lax.rsqrt(var + _EPS)
            a2c2_ref[0] = a2
            a2c2_ref[1] = b2_ref[...] - meanz * a2

        for b in range(bb):
            z = jnp.dot(w2_ref[...], hbuf_ref[i * bb + b],
                        preferred_element_type=jnp.float32)
            z = z * a2c2_ref[0] + a2c2_ref[1]
            y_ref[b] = z.reshape(z.shape[0], hw // ww, ww)


@jax.jit
def _dsconv(x, w_dw, g1, b1, w_pw, g2, b2):
    n_img, cin, hh, ww = x.shape
    kk = w_dw.shape[-1]
    assert kk == 3 and kk // 2 == 1
    cout = w_pw.shape[0]
    hw = hh * ww
    padhw = hw + 4 * ww
    inv_count = 1.0 / float(n_img * hw)
    bb = 2
    assert n_img % bb == 0
    nsteps = n_img // bb

    x2 = x.astype(jnp.float32)
    wdw9 = w_dw.reshape(cin, kk * kk).T.reshape(kk * kk, cin, 1)
    wdw9 = wdw9.astype(jnp.float32)
    w2 = w_pw.reshape(cout, cin).astype(jnp.bfloat16)
    g1r = g1.reshape(cin, 1).astype(jnp.float32)
    b1r = b1.reshape(cin, 1).astype(jnp.float32)
    g2r = g2.reshape(cout, 1).astype(jnp.float32)
    b2r = b2.reshape(cout, 1).astype(jnp.float32)

    y = pl.pallas_call(
        functools.partial(_fused_kernel, bb=bb, cin=cin, ww=ww, hw=hw,
                          padhw=padhw, inv_count=inv_count),
        grid=(3, nsteps),
        in_specs=[
            pl.BlockSpec((bb, cin, hh, ww),
                         lambda ph, i: (jnp.where(ph == 0, i, 0), 0, 0, 0)),
            pl.BlockSpec((kk * kk, cin, 1), lambda ph, i: (0, 0, 0)),
            pl.BlockSpec((cout, cin), lambda ph, i: (0, 0)),
            pl.BlockSpec((cin, 1), lambda ph, i: (0, 0)),
            pl.BlockSpec((cin, 1), lambda ph, i: (0, 0)),
            pl.BlockSpec((cout, 1), lambda ph, i: (0, 0)),
            pl.BlockSpec((cout, 1), lambda ph, i: (0, 0)),
        ],
        out_specs=pl.BlockSpec(
            (bb, cout, hh, ww),
            lambda ph, i: (jnp.where(ph == 2, i, 0), 0, 0, 0)),
        out_shape=jax.ShapeDtypeStruct((n_img, cout, hh, ww), jnp.float32),
        scratch_shapes=[
            pltpu.VMEM((bb, cin, padhw), jnp.float32),
            pltpu.VMEM((bb, cin, padhw), jnp.float32),
            pltpu.VMEM((bb, cin, padhw), jnp.float32),
            pltpu.VMEM((n_img, cin, hw), jnp.bfloat16),
            pltpu.VMEM((2, cin, 1), jnp.float32),
            pltpu.VMEM((2, cin, 1), jnp.float32),
            pltpu.VMEM((cin, cin), jnp.float32),
            pltpu.VMEM((cin, 1), jnp.float32),
            pltpu.VMEM((2, cout, 1), jnp.float32),
        ],
        compiler_params=pltpu.CompilerParams(
            dimension_semantics=("arbitrary", "arbitrary"),
            vmem_limit_bytes=58 * 1024 * 1024,
        ),
        cost_estimate=pl.CostEstimate(
            flops=2 * n_img * cin * hw * kk * kk
            + 2 * n_img * hw * cin * (cin + cout),
            transcendentals=0,
            bytes_accessed=4 * (x2.size + n_img * cout * hw),
        ),
    )(x2, wdw9, w2, g1r, b1r, g2r, b2r)

    return y


def kernel(x, w_dw, g1, b1, w_pw, g2, b2):
    return _dsconv(x, w_dw, g1, b1, w_pw, g2, b2)


# speedup vs baseline: 1.2827x; 1.2827x over previous
import functools

import jax
import jax.numpy as jnp
from jax import lax
from jax.experimental import pallas as pl
from jax.experimental.pallas import tpu as pltpu

_EPS = 1e-5


def _fused_kernel(x_ref, wdw_ref, w2_ref, g1_ref, b1_ref, g2_ref, b2_ref,
                  y_ref,
                  xf_ref, xa_ref, xb_ref, hbuf_ref, st1_ref, a1c1_ref,
                  ms_ref, ss_ref, a2c2_ref,
                  *, bb, cin, ww, hw, padhw, inv_count):
    ph = pl.program_id(0)
    i = pl.program_id(1)

    @pl.when(ph == 0)
    def _phase0():
        @pl.when(i == 0)
        def _init():
            xf_ref[...] = jnp.zeros_like(xf_ref)
            st1_ref[...] = jnp.zeros_like(st1_ref)

        for b in range(bb):
            xf_ref[b, :, 2 * ww:2 * ww + hw] = x_ref[b].reshape(cin, hw)
        xfv = xf_ref[...]
        lane = lax.broadcasted_iota(jnp.int32, (1, padhw), 1) % ww
        xa_ref[...] = jnp.where(lane == ww - 1, 0.0, xfv)
        xb_ref[...] = jnp.where(lane == 0, 0.0, xfv)
        st_s = None
        st_q = None
        for b in range(bb):
            acc = None
            for kh in range(3):
                for kw in range(3):
                    src = (xa_ref, xf_ref, xb_ref)[kw]
                    s = (ww - 1) + kh * ww + kw
                    p = jnp.dot(wdw_ref[kh * 3 + kw], src[b, :, s:s + hw],
                                preferred_element_type=jnp.float32)
                    acc = p if acc is None else acc + p
            hbuf_ref[i * bb + b] = acc
            ps = jnp.sum(acc, axis=1, keepdims=True)
            pq = jnp.sum(acc * acc, axis=1, keepdims=True)
            st_s = ps if st_s is None else st_s + ps
            st_q = pq if st_q is None else st_q + pq
        st1_ref[0] += st_s
        st1_ref[1] += st_q

    @pl.when(ph == 1)
    def _phase1():
        @pl.when(i == 0)
        def _fold_bn1():
            mean = st1_ref[0] * inv_count
            var = jnp.maximum(st1_ref[1] * inv_count - mean * mean, 0.0)
            a1 = g1_ref[...] * lax.rsqrt(var + _EPS)
            a1c1_ref[0] = a1
            a1c1_ref[1] = b1_ref[...] - mean * a1
            ms_ref[...] = jnp.zeros_like(ms_ref)
            ss_ref[...] = jnp.zeros_like(ss_ref)

        g = jnp.clip(hbuf_ref[pl.ds(i * bb, bb)] * a1c1_ref[0]
                     + a1c1_ref[1], 0.0, 6.0)
        hbuf_ref[pl.ds(i * bb, bb)] = g
        ss_ref[...] += jnp.sum(g, axis=(0, 2), keepdims=True)[0]
        gram = None
        for b in range(bb):
            gb = hbuf_ref[i * bb + b]
            p = lax.dot_general(gb, gb, (((1,), (1,)), ((), ())),
                                preferred_element_type=jnp.float32)
            gram = p if gram is None else gram + p
        ms_ref[...] += gram

    @pl.when(ph == 2)
    def _phase2():
        @pl.when(i == 0)
        def _fold_bn2():
            w2v = w2_ref[...]
            meanz = lax.dot_general(
                w2v, ss_ref[...], (((1,), (0,)), ((), ())),
                preferred_element_type=jnp.float32) * inv_count
            t = jnp.dot(w2v, ms_ref[...],
                        preferred_element_type=jnp.float32)
            ez2 = jnp.sum(t * w2v, axis=1, keepdims=True) * inv_count
            var = jnp.maximum(ez2 - meanz * meanz, 0.0)
            a2 = g2_ref[...] * lax.rsqrt(var + _EPS)
            a2c2_ref[0] = a2
            a2c2_ref[1] = b2_ref[...] - meanz * a2

        for b in range(bb):
            z = jnp.dot(w2_ref[...], hbuf_ref[i * bb + b],
                        preferred_element_type=jnp.float32)
            z = z * a2c2_ref[0] + a2c2_ref[1]
            y_ref[b] = z.reshape(z.shape[0], hw // ww, ww)


@jax.jit
def _dsconv(x, w_dw, g1, b1, w_pw, g2, b2):
    n_img, cin, hh, ww = x.shape
    kk = w_dw.shape[-1]
    assert kk == 3 and kk // 2 == 1
    cout = w_pw.shape[0]
    hw = hh * ww
    padhw = hw + 4 * ww
    inv_count = 1.0 / float(n_img * hw)
    bb = 2
    assert n_img % bb == 0
    nsteps = n_img // bb

    x2 = x.astype(jnp.float32)
    wdw9 = jax.vmap(jnp.diag)(
        w_dw.reshape(cin, kk * kk).T.astype(jnp.float32))
    w2 = w_pw.reshape(cout, cin).astype(jnp.float32)
    g1r = g1.reshape(cin, 1).astype(jnp.float32)
    b1r = b1.reshape(cin, 1).astype(jnp.float32)
    g2r = g2.reshape(cout, 1).astype(jnp.float32)
    b2r = b2.reshape(cout, 1).astype(jnp.float32)

    y = pl.pallas_call(
        functools.partial(_fused_kernel, bb=bb, cin=cin, ww=ww, hw=hw,
                          padhw=padhw, inv_count=inv_count),
        grid=(3, nsteps),
        in_specs=[
            pl.BlockSpec((bb, cin, hh, ww),
                         lambda ph, i: (jnp.where(ph == 0, i, 0), 0, 0, 0)),
            pl.BlockSpec((kk * kk, cin, cin), lambda ph, i: (0, 0, 0)),
            pl.BlockSpec((cout, cin), lambda ph, i: (0, 0)),
            pl.BlockSpec((cin, 1), lambda ph, i: (0, 0)),
            pl.BlockSpec((cin, 1), lambda ph, i: (0, 0)),
            pl.BlockSpec((cout, 1), lambda ph, i: (0, 0)),
            pl.BlockSpec((cout, 1), lambda ph, i: (0, 0)),
        ],
        out_specs=pl.BlockSpec(
            (bb, cout, hh, ww),
            lambda ph, i: (jnp.where(ph == 2, i, 0), 0, 0, 0)),
        out_shape=jax.ShapeDtypeStruct((n_img, cout, hh, ww), jnp.float32),
        scratch_shapes=[
            pltpu.VMEM((bb, cin, padhw), jnp.float32),
            pltpu.VMEM((bb, cin, padhw), jnp.float32),
            pltpu.VMEM((bb, cin, padhw), jnp.float32),
            pltpu.VMEM((n_img, cin, hw), jnp.float32),
            pltpu.VMEM((2, cin, 1), jnp.float32),
            pltpu.VMEM((2, cin, 1), jnp.float32),
            pltpu.VMEM((cin, cin), jnp.float32),
            pltpu.VMEM((cin, 1), jnp.float32),
            pltpu.VMEM((2, cout, 1), jnp.float32),
        ],
        compiler_params=pltpu.CompilerParams(
            dimension_semantics=("arbitrary", "arbitrary"),
            vmem_limit_bytes=58 * 1024 * 1024,
        ),
        cost_estimate=pl.CostEstimate(
            flops=2 * n_img * cin * hw * kk * kk
            + 2 * n_img * hw * cin * (cin + cout),
            transcendentals=0,
            bytes_accessed=4 * (x2.size + n_img * cout * hw),
        ),
    )(x2, wdw9, w2, g1r, b1r, g2r, b2r)

    return y


def kernel(x, w_dw, g1, b1, w_pw, g2, b2):
    return _dsconv(x, w_dw, g1, b1, w_pw, g2, b2)
